# Initial kernel scaffold; baseline (speedup 1.0000x reference)
import jax
import jax.numpy as jnp
from jax.experimental import pallas as pl


def kernel(embedding, product1, product2):
    raise NotImplementedError("write your pallas kernel here")


# speedup vs baseline: 1.7978x; 1.7978x over previous
import jax
import jax.numpy as jnp
from jax import lax
from jax.experimental import pallas as pl
from jax.experimental.pallas import tpu as pltpu


def _round_up(x, m):
    return ((x + m - 1) // m) * m


def _mf_kernel(tab_ref, p1_ref, p2_ref, out_ref):
    tab = tab_ref[...]
    E = tab.shape[1]
    tb = p1_ref.shape[1]
    n_hi = E // 128

    hi_iota = lax.broadcasted_iota(jnp.int32, (n_hi, 1, tb), 0)
    lo_iota = lax.broadcasted_iota(jnp.int32, (1, 128, tb), 1)

    def gather_t(p_ref):
        p = p_ref[...].reshape(1, 1, tb)
        a = (hi_iota == (p >> 7)).astype(jnp.bfloat16)
        b = (lo_iota == (p & 127)).astype(jnp.bfloat16)
        oh = (a * b).reshape(E, tb)
        return jnp.dot(tab, oh, preferred_element_type=jnp.float32)

    e1t = gather_t(p1_ref)
    e2t = gather_t(p2_ref)
    out_ref[...] = jnp.sum(e1t * e2t, axis=0, keepdims=True)


def kernel(embedding, product1, product2, *, tb=1024):
    E, D = embedding.shape
    B = product1.shape[0]
    assert E % 128 == 0

    tab = embedding.T.astype(jnp.bfloat16)
    tb = min(tb, _round_up(B, 128))
    padded_b = _round_up(B, tb)

    p1 = jnp.zeros((1, padded_b), jnp.int32).at[0, :B].set(
        product1.astype(jnp.int32))
    p2 = jnp.zeros((1, padded_b), jnp.int32).at[0, :B].set(
        product2.astype(jnp.int32))

    out = pl.pallas_call(
        _mf_kernel,
        out_shape=jax.ShapeDtypeStruct((1, padded_b), jnp.float32),
        grid_spec=pltpu.PrefetchScalarGridSpec(
            num_scalar_prefetch=0,
            grid=(padded_b // tb,),
            in_specs=[
                pl.BlockSpec((D, E), lambda i: (0, 0)),
                pl.BlockSpec((1, tb), lambda i: (0, i)),
                pl.BlockSpec((1, tb), lambda i: (0, i)),
            ],
            out_specs=pl.BlockSpec((1, tb), lambda i: (0, i)),
        ),
        compiler_params=pltpu.CompilerParams(
            dimension_semantics=("parallel",),
            vmem_limit_bytes=40 * 2**20,
        ),
    )(tab, p1, p2)
    return out[0, :B]


# speedup vs baseline: 1.9956x; 1.1100x over previous
import jax
import jax.numpy as jnp
from jax import lax
from jax.experimental import pallas as pl
from jax.experimental.pallas import tpu as pltpu


def _round_up(x, m):
    return ((x + m - 1) // m) * m


def _mf_kernel(tab_ref, p1_ref, p2_ref, out_ref):
    tab = tab_ref[...]
    E = tab.shape[1]
    tb = p1_ref.shape[1]
    n_hi = E // 128

    hi_iota = lax.broadcasted_iota(jnp.int32, (n_hi, 1, tb), 0)
    lo_iota = lax.broadcasted_iota(jnp.int32, (1, 128, tb), 1)

    def gather_t(p_ref):
        p = p_ref[...].reshape(1, 1, tb)
        a = (hi_iota == (p >> 7)).astype(jnp.bfloat16)
        b = (lo_iota == (p & 127)).astype(jnp.bfloat16)
        oh = (a * b).reshape(E, tb)
        return jnp.dot(tab, oh, preferred_element_type=jnp.float32)

    e1t = gather_t(p1_ref)
    e2t = gather_t(p2_ref)
    out_ref[...] = jnp.sum(e1t * e2t, axis=0, keepdims=True)


def kernel(embedding, product1, product2, *, tb=2048):
    E, D = embedding.shape
    B = product1.shape[0]
    assert E % 128 == 0

    tab = embedding.T.astype(jnp.bfloat16)
    tb = min(tb, _round_up(B, 128))
    padded_b = _round_up(B, tb)

    p1 = jnp.zeros((1, padded_b), jnp.int32).at[0, :B].set(
        product1.astype(jnp.int32))
    p2 = jnp.zeros((1, padded_b), jnp.int32).at[0, :B].set(
        product2.astype(jnp.int32))

    out = pl.pallas_call(
        _mf_kernel,
        out_shape=jax.ShapeDtypeStruct((1, padded_b), jnp.float32),
        grid_spec=pltpu.PrefetchScalarGridSpec(
            num_scalar_prefetch=0,
            grid=(padded_b // tb,),
            in_specs=[
                pl.BlockSpec((D, E), lambda i: (0, 0)),
                pl.BlockSpec((1, tb), lambda i: (0, i)),
                pl.BlockSpec((1, tb), lambda i: (0, i)),
            ],
            out_specs=pl.BlockSpec((1, tb), lambda i: (0, i)),
        ),
        compiler_params=pltpu.CompilerParams(
            dimension_semantics=("parallel",),
            vmem_limit_bytes=40 * 2**20,
        ),
    )(tab, p1, p2)
    return out[0, :B]


# speedup vs baseline: 2.1106x; 1.0576x over previous
import jax
import jax.numpy as jnp
from jax import lax
from jax.experimental import pallas as pl
from jax.experimental.pallas import tpu as pltpu


def _round_up(x, m):
    return ((x + m - 1) // m) * m


def _mf_kernel(tab_ref, p1_ref, p2_ref, out_ref):
    tab = tab_ref[...]
    E = tab.shape[1]
    tb = p1_ref.shape[1]
    n_hi = E // 128

    hi_iota = lax.broadcasted_iota(jnp.int32, (n_hi, 1, tb), 0)
    lo_iota = lax.broadcasted_iota(jnp.int32, (1, 128, tb), 1)

    def gather_t(p_ref):
        p = p_ref[...].reshape(1, 1, tb)
        a = (hi_iota == (p >> 7)).astype(jnp.bfloat16)
        b = (lo_iota == (p & 127)).astype(jnp.bfloat16)
        oh = (a * b).reshape(E, tb)
        return jnp.dot(tab, oh, preferred_element_type=jnp.float32)

    e1t = gather_t(p1_ref)
    e2t = gather_t(p2_ref)
    out_ref[...] = jnp.sum(e1t * e2t, axis=0, keepdims=True)


def kernel(embedding, product1, product2, *, tb=4096):
    E, D = embedding.shape
    B = product1.shape[0]
    assert E % 128 == 0

    tab = embedding.T.astype(jnp.bfloat16)
    tb = min(tb, _round_up(B, 128))
    padded_b = _round_up(B, tb)

    p1 = jnp.zeros((1, padded_b), jnp.int32).at[0, :B].set(
        product1.astype(jnp.int32))
    p2 = jnp.zeros((1, padded_b), jnp.int32).at[0, :B].set(
        product2.astype(jnp.int32))

    out = pl.pallas_call(
        _mf_kernel,
        out_shape=jax.ShapeDtypeStruct((1, padded_b), jnp.float32),
        grid_spec=pltpu.PrefetchScalarGridSpec(
            num_scalar_prefetch=0,
            grid=(padded_b // tb,),
            in_specs=[
                pl.BlockSpec((D, E), lambda i: (0, 0)),
                pl.BlockSpec((1, tb), lambda i: (0, i)),
                pl.BlockSpec((1, tb), lambda i: (0, i)),
            ],
            out_specs=pl.BlockSpec((1, tb), lambda i: (0, i)),
        ),
        compiler_params=pltpu.CompilerParams(
            dimension_semantics=("parallel",),
            vmem_limit_bytes=56 * 2**20,
        ),
    )(tab, p1, p2)
    return out[0, :B]


# speedup vs baseline: 2.1704x; 1.0283x over previous
import jax
import jax.numpy as jnp
from jax import lax
from jax.experimental import pallas as pl
from jax.experimental.pallas import tpu as pltpu


def _round_up(x, m):
    return ((x + m - 1) // m) * m


def _mf_kernel(tab_ref, p1_ref, p2_ref, out_ref):
    tab = tab_ref[...]
    E = tab.shape[1]
    tb = p1_ref.shape[1]
    n_hi = E // 128

    hi_iota = lax.broadcasted_iota(jnp.int32, (n_hi, 1, tb), 0)
    lo_iota = lax.broadcasted_iota(jnp.int32, (1, 128, tb), 1)

    def gather_t(p_ref):
        p = p_ref[...].reshape(1, 1, tb)
        a = (hi_iota == (p >> 7)).astype(jnp.bfloat16)
        b = (lo_iota == (p & 127)).astype(jnp.bfloat16)
        oh = (a * b).reshape(E, tb)
        return jnp.dot(tab, oh, preferred_element_type=jnp.float32)

    e1t = gather_t(p1_ref)
    e2t = gather_t(p2_ref)
    out_ref[...] = jnp.sum(e1t * e2t, axis=0, keepdims=True)


def kernel(embedding, product1, product2, *, tb=8192):
    E, D = embedding.shape
    B = product1.shape[0]
    assert E % 128 == 0

    tab = embedding.T.astype(jnp.bfloat16)
    tb = min(tb, _round_up(B, 128))
    padded_b = _round_up(B, tb)

    p1 = jnp.zeros((1, padded_b), jnp.int32).at[0, :B].set(
        product1.astype(jnp.int32))
    p2 = jnp.zeros((1, padded_b), jnp.int32).at[0, :B].set(
        product2.astype(jnp.int32))

    out = pl.pallas_call(
        _mf_kernel,
        out_shape=jax.ShapeDtypeStruct((1, padded_b), jnp.float32),
        grid_spec=pltpu.PrefetchScalarGridSpec(
            num_scalar_prefetch=0,
            grid=(padded_b // tb,),
            in_specs=[
                pl.BlockSpec((D, E), lambda i: (0, 0)),
                pl.BlockSpec((1, tb), lambda i: (0, i)),
                pl.BlockSpec((1, tb), lambda i: (0, i)),
            ],
            out_specs=pl.BlockSpec((1, tb), lambda i: (0, i)),
        ),
        compiler_params=pltpu.CompilerParams(
            dimension_semantics=("parallel",),
            vmem_limit_bytes=56 * 2**20,
        ),
    )(tab, p1, p2)
    return out[0, :B]
